# Initial kernel scaffold; baseline (speedup 1.0000x reference)
import jax
import jax.numpy as jnp
from jax.experimental import pallas as pl


def kernel(input_ids, emb_table, Wih_l0_d0, Whh_l0_d0, bih_l0_d0, bhh_l0_d0, Wih_l0_d1, Whh_l0_d1, bih_l0_d1, bhh_l0_d1, Wih_l1_d0, Whh_l1_d0, bih_l1_d0, bhh_l1_d0, Wih_l1_d1, Whh_l1_d1, bih_l1_d1, bhh_l1_d1, fc_w, fc_b):
    raise NotImplementedError("write your pallas kernel here")


# speedup vs baseline: 12.7223x; 12.7223x over previous
import functools

import jax
import jax.numpy as jnp
from jax.experimental import pallas as pl
from jax.experimental.pallas import tpu as pltpu
from jax.experimental.pallas import tpu_sc as plsc

V, D, H2, L, T = 30000, 256, 512, 2, 4
H = H2 // 2
B, S = 32, 512
G4 = 4 * H
M = B * S

_WIN = 128


def _emb_gather(table, ids_flat):
    mesh = plsc.VectorSubcoreMesh(core_axis_name="core",
                                  subcore_axis_name="subcore")
    idx2 = ids_flat.reshape(1, M)

    @functools.partial(
        pl.kernel,
        out_type=jax.ShapeDtypeStruct((M, D), jnp.float32),
        mesh=mesh,
    )
    def k(tab_hbm, i_hbm, o_hbm):
        def body(i_vmem, o_vmem):
            pltpu.sync_copy(tab_hbm.at[i_vmem.at[0]], o_vmem)

        pltpu.emit_pipeline(
            body,
            grid=(M // _WIN,),
            in_specs=[pl.BlockSpec((1, _WIN), index_map=lambda i: (0, i))],
            out_specs=[pl.BlockSpec((_WIN, D), index_map=lambda i: (i, 0))],
            core_axis_name="subcore",
            dimension_semantics=(pltpu.PARALLEL,),
        )(i_hbm, o_hbm)

    return k(table, idx2)


_MB = 1024


def _mm1_body(x_ref, w_ref, b_ref, o_ref):
    o_ref[...] = (
        jnp.dot(x_ref[...], w_ref[...], preferred_element_type=jnp.float32)
        + b_ref[...]
    )


def _mm1(x, w, b):
    m, k = x.shape
    n = w.shape[1]
    return pl.pallas_call(
        _mm1_body,
        grid=(m // _MB,),
        in_specs=[
            pl.BlockSpec((_MB, k), lambda i: (i, 0)),
            pl.BlockSpec((k, n), lambda i: (0, 0)),
            pl.BlockSpec((1, n), lambda i: (0, 0)),
        ],
        out_specs=pl.BlockSpec((_MB, n), lambda i: (i, 0)),
        out_shape=jax.ShapeDtypeStruct((m, n), jnp.float32),
        compiler_params=pltpu.CompilerParams(
            dimension_semantics=("parallel",)),
    )(x, w, b.reshape(1, n))


def _mm2_body(xa_ref, xb_ref, wa_ref, wb_ref, b_ref, o_ref):
    acc = jnp.dot(xa_ref[...], wa_ref[...], preferred_element_type=jnp.float32)
    acc = acc + jnp.dot(xb_ref[...], wb_ref[...],
                        preferred_element_type=jnp.float32)
    o_ref[...] = acc + b_ref[...]


def _mm2(xa, xb, wa, wb, b):
    m, ka = xa.shape
    kb = xb.shape[1]
    n = wa.shape[1]
    return pl.pallas_call(
        _mm2_body,
        grid=(m // _MB,),
        in_specs=[
            pl.BlockSpec((_MB, ka), lambda i: (i, 0)),
            pl.BlockSpec((_MB, kb), lambda i: (i, 0)),
            pl.BlockSpec((ka, n), lambda i: (0, 0)),
            pl.BlockSpec((kb, n), lambda i: (0, 0)),
            pl.BlockSpec((1, n), lambda i: (0, 0)),
        ],
        out_specs=pl.BlockSpec((_MB, n), lambda i: (i, 0)),
        out_shape=jax.ShapeDtypeStruct((m, n), jnp.float32),
        compiler_params=pltpu.CompilerParams(
            dimension_semantics=("parallel",)),
    )(xa, xb, wa, wb, b.reshape(1, n))


_CHUNK = 8
_NBLK = S // _CHUNK


def _lstm_step(x_gates, h, c, w):
    g = x_gates + jnp.dot(h, w, preferred_element_type=jnp.float32)
    sg = jax.nn.sigmoid(g[:, : 3 * H])
    gg = jnp.tanh(g[:, 3 * H:])
    c2 = sg[:, H: 2 * H] * c + sg[:, :H] * gg
    h2 = sg[:, 2 * H: 3 * H] * jnp.tanh(c2)
    return h2, c2


def _scan_body(xf_ref, xb_ref, wf_ref, wb_ref, of_ref, ob_ref,
               hf_ref, cf_ref, hb_ref, cb_ref):
    @pl.when(pl.program_id(0) == 0)
    def _():
        z = jnp.zeros((B, H), jnp.float32)
        hf_ref[...] = z
        cf_ref[...] = z
        hb_ref[...] = z
        cb_ref[...] = z

    wf = wf_ref[...]
    wb = wb_ref[...]
    for j in range(_CHUNK):
        h2, c2 = _lstm_step(xf_ref[j], hf_ref[...], cf_ref[...], wf)
        hf_ref[...] = h2
        cf_ref[...] = c2
        of_ref[j] = h2
        jb = _CHUNK - 1 - j
        h2, c2 = _lstm_step(xb_ref[jb], hb_ref[...], cb_ref[...], wb)
        hb_ref[...] = h2
        cb_ref[...] = c2
        ob_ref[jb] = h2


def _bilstm_scan(xp, wfT, wbT):
    return pl.pallas_call(
        _scan_body,
        grid=(_NBLK,),
        in_specs=[
            pl.BlockSpec((_CHUNK, B, G4), lambda i: (i, 0, 0)),
            pl.BlockSpec((_CHUNK, B, G4), lambda i: (_NBLK - 1 - i, 0, 1)),
            pl.BlockSpec((H, G4), lambda i: (0, 0)),
            pl.BlockSpec((H, G4), lambda i: (0, 0)),
        ],
        out_specs=[
            pl.BlockSpec((_CHUNK, B, H), lambda i: (i, 0, 0)),
            pl.BlockSpec((_CHUNK, B, H), lambda i: (_NBLK - 1 - i, 0, 0)),
        ],
        out_shape=[
            jax.ShapeDtypeStruct((S, B, H), jnp.float32),
            jax.ShapeDtypeStruct((S, B, H), jnp.float32),
        ],
        scratch_shapes=[pltpu.VMEM((B, H), jnp.float32) for _ in range(4)],
        compiler_params=pltpu.CompilerParams(
            dimension_semantics=("arbitrary",)),
    )(xp, xp, wfT, wbT)


def _perm_rows(w):
    return jnp.concatenate([w[: 2 * H], w[3 * H:], w[2 * H: 3 * H]], axis=0)


def kernel(input_ids, emb_table, Wih_l0_d0, Whh_l0_d0, bih_l0_d0, bhh_l0_d0,
           Wih_l0_d1, Whh_l0_d1, bih_l0_d1, bhh_l0_d1, Wih_l1_d0, Whh_l1_d0,
           bih_l1_d0, bhh_l1_d0, Wih_l1_d1, Whh_l1_d1, bih_l1_d1, bhh_l1_d1,
           fc_w, fc_b):
    wih = {}
    whh = {}
    bias = {}
    params = {
        (0, 0): (Wih_l0_d0, Whh_l0_d0, bih_l0_d0, bhh_l0_d0),
        (0, 1): (Wih_l0_d1, Whh_l0_d1, bih_l0_d1, bhh_l0_d1),
        (1, 0): (Wih_l1_d0, Whh_l1_d0, bih_l1_d0, bhh_l1_d0),
        (1, 1): (Wih_l1_d1, Whh_l1_d1, bih_l1_d1, bhh_l1_d1),
    }
    for (l, d), (wi, wh, bi, bh) in params.items():
        wih[(l, d)] = _perm_rows(wi).T
        whh[(l, d)] = _perm_rows(wh).T
        bias[(l, d)] = _perm_rows((bi + bh).reshape(G4, 1)).reshape(G4)

    ids_tm = input_ids.T.reshape(M).astype(jnp.int32)
    x = _emb_gather(emb_table, ids_tm)

    w0 = jnp.concatenate([wih[(0, 0)], wih[(0, 1)]], axis=1)
    b0 = jnp.concatenate([bias[(0, 0)], bias[(0, 1)]])
    xp0 = _mm1(x, w0, b0).reshape(S, B, 2 * G4)
    hf0, hb0 = _bilstm_scan(xp0, whh[(0, 0)], whh[(0, 1)])

    w1a = jnp.concatenate([wih[(1, 0)][:H], wih[(1, 1)][:H]], axis=1)
    w1b = jnp.concatenate([wih[(1, 0)][H:], wih[(1, 1)][H:]], axis=1)
    b1 = jnp.concatenate([bias[(1, 0)], bias[(1, 1)]])
    xp1 = _mm2(hf0.reshape(M, H), hb0.reshape(M, H), w1a, w1b, b1)
    hf1, hb1 = _bilstm_scan(xp1.reshape(S, B, 2 * G4),
                            whh[(1, 0)], whh[(1, 1)])

    f = fc_w.reshape(T * 5, H2).T
    fpad = jnp.zeros((H2, 128), jnp.float32).at[:, : T * 5].set(f)
    bpad = jnp.zeros((128,), jnp.float32).at[: T * 5].set(fc_b.reshape(T * 5))
    y = _mm2(hf1.reshape(M, H), hb1.reshape(M, H),
             fpad[:H], fpad[H:], bpad)
    logits = y[:, : T * 5].reshape(S, B, T, 5).transpose(1, 2, 0, 3)
    return logits


# speedup vs baseline: 14.0073x; 1.1010x over previous
import functools

import jax
import jax.numpy as jnp
from jax.experimental import pallas as pl
from jax.experimental.pallas import tpu as pltpu
from jax.experimental.pallas import tpu_sc as plsc

V, D, H2, L, T = 30000, 256, 512, 2, 4
H = H2 // 2
B, S = 32, 512
G4 = 4 * H
M = B * S

_WIN = 128


def _emb_gather(table, ids_flat):
    mesh = plsc.VectorSubcoreMesh(core_axis_name="core",
                                  subcore_axis_name="subcore")
    idx2 = ids_flat.reshape(1, M)

    @functools.partial(
        pl.kernel,
        out_type=jax.ShapeDtypeStruct((M, D), jnp.float32),
        mesh=mesh,
    )
    def k(tab_hbm, i_hbm, o_hbm):
        def body(i_vmem, o_vmem):
            pltpu.sync_copy(tab_hbm.at[i_vmem.at[0]], o_vmem)

        pltpu.emit_pipeline(
            body,
            grid=(M // _WIN,),
            in_specs=[pl.BlockSpec((1, _WIN), index_map=lambda i: (0, i))],
            out_specs=[pl.BlockSpec((_WIN, D), index_map=lambda i: (i, 0))],
            core_axis_name="subcore",
            dimension_semantics=(pltpu.PARALLEL,),
        )(i_hbm, o_hbm)

    return k(table, idx2)


_MB = 1024


def _mm1_body(x_ref, w_ref, b_ref, o_ref):
    acc = jnp.dot(x_ref[...], w_ref[...], preferred_element_type=jnp.float32)
    o_ref[...] = (acc + b_ref[...]).astype(o_ref.dtype)


def _mm1(x, w, b, out_dtype):
    m, k = x.shape
    n = w.shape[1]
    return pl.pallas_call(
        _mm1_body,
        grid=(m // _MB,),
        in_specs=[
            pl.BlockSpec((_MB, k), lambda i: (i, 0)),
            pl.BlockSpec((k, n), lambda i: (0, 0)),
            pl.BlockSpec((1, n), lambda i: (0, 0)),
        ],
        out_specs=pl.BlockSpec((_MB, n), lambda i: (i, 0)),
        out_shape=jax.ShapeDtypeStruct((m, n), out_dtype),
        compiler_params=pltpu.CompilerParams(
            dimension_semantics=("parallel",)),
    )(x, w, b.reshape(1, n))


def _mm2_body(xa_ref, xb_ref, wa_ref, wb_ref, b_ref, o_ref):
    acc = jnp.dot(xa_ref[...], wa_ref[...], preferred_element_type=jnp.float32)
    acc = acc + jnp.dot(xb_ref[...], wb_ref[...],
                        preferred_element_type=jnp.float32)
    o_ref[...] = (acc + b_ref[...]).astype(o_ref.dtype)


def _mm2(xa, xb, wa, wb, b, out_dtype):
    m, ka = xa.shape
    kb = xb.shape[1]
    n = wa.shape[1]
    return pl.pallas_call(
        _mm2_body,
        grid=(m // _MB,),
        in_specs=[
            pl.BlockSpec((_MB, ka), lambda i: (i, 0)),
            pl.BlockSpec((_MB, kb), lambda i: (i, 0)),
            pl.BlockSpec((ka, n), lambda i: (0, 0)),
            pl.BlockSpec((kb, n), lambda i: (0, 0)),
            pl.BlockSpec((1, n), lambda i: (0, 0)),
        ],
        out_specs=pl.BlockSpec((_MB, n), lambda i: (i, 0)),
        out_shape=jax.ShapeDtypeStruct((m, n), out_dtype),
        compiler_params=pltpu.CompilerParams(
            dimension_semantics=("parallel",)),
    )(xa, xb, wa, wb, b.reshape(1, n))


_CHUNK = 8
_NBLK = S // _CHUNK


def _lstm_step(x_gates, h, c, w):
    g = x_gates.astype(jnp.float32) + jnp.dot(
        h.astype(jnp.bfloat16), w, preferred_element_type=jnp.float32)
    sg = jax.nn.sigmoid(g[:, : 3 * H])
    gg = jnp.tanh(g[:, 3 * H:])
    c2 = sg[:, H: 2 * H] * c + sg[:, :H] * gg
    h2 = sg[:, 2 * H: 3 * H] * jnp.tanh(c2)
    return h2, c2


def _scan_body(xf_ref, xb_ref, wf_ref, wb_ref, of_ref, ob_ref,
               hf_ref, cf_ref, hb_ref, cb_ref):
    @pl.when(pl.program_id(0) == 0)
    def _():
        z = jnp.zeros((B, H), jnp.float32)
        hf_ref[...] = z
        cf_ref[...] = z
        hb_ref[...] = z
        cb_ref[...] = z

    wf = wf_ref[...]
    wb = wb_ref[...]
    for j in range(_CHUNK):
        h2, c2 = _lstm_step(xf_ref[j], hf_ref[...], cf_ref[...], wf)
        hf_ref[...] = h2
        cf_ref[...] = c2
        of_ref[j] = h2.astype(jnp.bfloat16)
        jb = _CHUNK - 1 - j
        h2, c2 = _lstm_step(xb_ref[jb], hb_ref[...], cb_ref[...], wb)
        hb_ref[...] = h2
        cb_ref[...] = c2
        ob_ref[jb] = h2.astype(jnp.bfloat16)


def _bilstm_scan(xp, wfT, wbT):
    return pl.pallas_call(
        _scan_body,
        grid=(_NBLK,),
        in_specs=[
            pl.BlockSpec((_CHUNK, B, G4), lambda i: (i, 0, 0)),
            pl.BlockSpec((_CHUNK, B, G4), lambda i: (_NBLK - 1 - i, 0, 1)),
            pl.BlockSpec((H, G4), lambda i: (0, 0)),
            pl.BlockSpec((H, G4), lambda i: (0, 0)),
        ],
        out_specs=[
            pl.BlockSpec((_CHUNK, B, H), lambda i: (i, 0, 0)),
            pl.BlockSpec((_CHUNK, B, H), lambda i: (_NBLK - 1 - i, 0, 0)),
        ],
        out_shape=[
            jax.ShapeDtypeStruct((S, B, H), jnp.bfloat16),
            jax.ShapeDtypeStruct((S, B, H), jnp.bfloat16),
        ],
        scratch_shapes=[pltpu.VMEM((B, H), jnp.float32) for _ in range(4)],
        compiler_params=pltpu.CompilerParams(
            dimension_semantics=("arbitrary",)),
    )(xp, xp, wfT, wbT)


def _perm_rows(w):
    return jnp.concatenate([w[: 2 * H], w[3 * H:], w[2 * H: 3 * H]], axis=0)


def kernel(input_ids, emb_table, Wih_l0_d0, Whh_l0_d0, bih_l0_d0, bhh_l0_d0,
           Wih_l0_d1, Whh_l0_d1, bih_l0_d1, bhh_l0_d1, Wih_l1_d0, Whh_l1_d0,
           bih_l1_d0, bhh_l1_d0, Wih_l1_d1, Whh_l1_d1, bih_l1_d1, bhh_l1_d1,
           fc_w, fc_b):
    wih = {}
    whh = {}
    bias = {}
    params = {
        (0, 0): (Wih_l0_d0, Whh_l0_d0, bih_l0_d0, bhh_l0_d0),
        (0, 1): (Wih_l0_d1, Whh_l0_d1, bih_l0_d1, bhh_l0_d1),
        (1, 0): (Wih_l1_d0, Whh_l1_d0, bih_l1_d0, bhh_l1_d0),
        (1, 1): (Wih_l1_d1, Whh_l1_d1, bih_l1_d1, bhh_l1_d1),
    }
    for (l, d), (wi, wh, bi, bh) in params.items():
        wih[(l, d)] = _perm_rows(wi).T.astype(jnp.bfloat16)
        whh[(l, d)] = _perm_rows(wh).T.astype(jnp.bfloat16)
        bias[(l, d)] = _perm_rows((bi + bh).reshape(G4, 1)).reshape(G4)

    ids_tm = input_ids.T.reshape(M).astype(jnp.int32)
    x = _emb_gather(emb_table, ids_tm)

    w0 = jnp.concatenate([wih[(0, 0)], wih[(0, 1)]], axis=1)
    b0 = jnp.concatenate([bias[(0, 0)], bias[(0, 1)]])
    xp0 = _mm1(x.astype(jnp.bfloat16), w0, b0,
               jnp.bfloat16).reshape(S, B, 2 * G4)
    hf0, hb0 = _bilstm_scan(xp0, whh[(0, 0)], whh[(0, 1)])

    w1a = jnp.concatenate([wih[(1, 0)][:H], wih[(1, 1)][:H]], axis=1)
    w1b = jnp.concatenate([wih[(1, 0)][H:], wih[(1, 1)][H:]], axis=1)
    b1 = jnp.concatenate([bias[(1, 0)], bias[(1, 1)]])
    xp1 = _mm2(hf0.reshape(M, H), hb0.reshape(M, H), w1a, w1b, b1,
               jnp.bfloat16)
    hf1, hb1 = _bilstm_scan(xp1.reshape(S, B, 2 * G4),
                            whh[(1, 0)], whh[(1, 1)])

    f = fc_w.reshape(T * 5, H2).T
    fpad = jnp.zeros((H2, 128), jnp.float32).at[:, : T * 5].set(f)
    fpad = fpad.astype(jnp.bfloat16)
    bpad = jnp.zeros((128,), jnp.float32).at[: T * 5].set(fc_b.reshape(T * 5))
    y = _mm2(hf1.reshape(M, H), hb1.reshape(M, H),
             fpad[:H], fpad[H:], bpad, jnp.float32)
    logits = y[:, : T * 5].reshape(S, B, T, 5).transpose(1, 2, 0, 3)
    return logits


# speedup vs baseline: 14.4148x; 1.0291x over previous
import functools

import jax
import jax.numpy as jnp
from jax.experimental import pallas as pl
from jax.experimental.pallas import tpu as pltpu
from jax.experimental.pallas import tpu_sc as plsc

V, D, H2, L, T = 30000, 256, 512, 2, 4
H = H2 // 2
B, S = 32, 512
G4 = 4 * H
M = B * S

_WIN = 128


def _emb_gather(table, ids_flat):
    mesh = plsc.VectorSubcoreMesh(core_axis_name="core",
                                  subcore_axis_name="subcore")
    idx2 = ids_flat.reshape(1, M)

    @functools.partial(
        pl.kernel,
        out_type=jax.ShapeDtypeStruct((M, D), jnp.float32),
        mesh=mesh,
    )
    def k(tab_hbm, i_hbm, o_hbm):
        def body(i_vmem, o_vmem):
            pltpu.sync_copy(tab_hbm.at[i_vmem.at[0]], o_vmem)

        pltpu.emit_pipeline(
            body,
            grid=(M // _WIN,),
            in_specs=[pl.BlockSpec((1, _WIN), index_map=lambda i: (0, i))],
            out_specs=[pl.BlockSpec((_WIN, D), index_map=lambda i: (i, 0))],
            core_axis_name="subcore",
            dimension_semantics=(pltpu.PARALLEL,),
        )(i_hbm, o_hbm)

    return k(table, idx2)


_MB = 1024


def _mm1_body(x_ref, w_ref, b_ref, o_ref):
    acc = jnp.dot(x_ref[...], w_ref[...], preferred_element_type=jnp.float32)
    o_ref[...] = (acc + b_ref[...]).astype(o_ref.dtype)


def _mm1(x, w, b, out_dtype):
    m, k = x.shape
    n = w.shape[1]
    return pl.pallas_call(
        _mm1_body,
        grid=(m // _MB,),
        in_specs=[
            pl.BlockSpec((_MB, k), lambda i: (i, 0)),
            pl.BlockSpec((k, n), lambda i: (0, 0)),
            pl.BlockSpec((1, n), lambda i: (0, 0)),
        ],
        out_specs=pl.BlockSpec((_MB, n), lambda i: (i, 0)),
        out_shape=jax.ShapeDtypeStruct((m, n), out_dtype),
        compiler_params=pltpu.CompilerParams(
            dimension_semantics=("parallel",)),
    )(x, w, b.reshape(1, n))


def _mm2_body(xa_ref, xb_ref, wa_ref, wb_ref, b_ref, o_ref):
    acc = jnp.dot(xa_ref[...], wa_ref[...], preferred_element_type=jnp.float32)
    acc = acc + jnp.dot(xb_ref[...], wb_ref[...],
                        preferred_element_type=jnp.float32)
    o_ref[...] = (acc + b_ref[...]).astype(o_ref.dtype)


def _mm2(xa, xb, wa, wb, b, out_dtype):
    m, ka = xa.shape
    kb = xb.shape[1]
    n = wa.shape[1]
    return pl.pallas_call(
        _mm2_body,
        grid=(m // _MB,),
        in_specs=[
            pl.BlockSpec((_MB, ka), lambda i: (i, 0)),
            pl.BlockSpec((_MB, kb), lambda i: (i, 0)),
            pl.BlockSpec((ka, n), lambda i: (0, 0)),
            pl.BlockSpec((kb, n), lambda i: (0, 0)),
            pl.BlockSpec((1, n), lambda i: (0, 0)),
        ],
        out_specs=pl.BlockSpec((_MB, n), lambda i: (i, 0)),
        out_shape=jax.ShapeDtypeStruct((m, n), out_dtype),
        compiler_params=pltpu.CompilerParams(
            dimension_semantics=("parallel",)),
    )(xa, xb, wa, wb, b.reshape(1, n))


_CHUNK = 16
_NBLK = S // _CHUNK


def _lstm_step(x_gates, h, c, w):
    g = x_gates.astype(jnp.float32) + jnp.dot(
        h.astype(jnp.bfloat16), w, preferred_element_type=jnp.float32)
    sg = jax.nn.sigmoid(g[:, : 3 * H])
    gg = jnp.tanh(g[:, 3 * H:])
    c2 = sg[:, H: 2 * H] * c + sg[:, :H] * gg
    h2 = sg[:, 2 * H: 3 * H] * jnp.tanh(c2)
    return h2, c2


def _scan_body(xf_ref, xb_ref, wf_ref, wb_ref, of_ref, ob_ref,
               hf_ref, cf_ref, hb_ref, cb_ref):
    @pl.when(pl.program_id(0) == 0)
    def _():
        z = jnp.zeros((B, H), jnp.float32)
        hf_ref[...] = z
        cf_ref[...] = z
        hb_ref[...] = z
        cb_ref[...] = z

    wf = wf_ref[...]
    wb = wb_ref[...]
    for j in range(_CHUNK):
        h2, c2 = _lstm_step(xf_ref[j], hf_ref[...], cf_ref[...], wf)
        hf_ref[...] = h2
        cf_ref[...] = c2
        of_ref[j] = h2.astype(jnp.bfloat16)
        jb = _CHUNK - 1 - j
        h2, c2 = _lstm_step(xb_ref[jb], hb_ref[...], cb_ref[...], wb)
        hb_ref[...] = h2
        cb_ref[...] = c2
        ob_ref[jb] = h2.astype(jnp.bfloat16)


def _bilstm_scan(xp, wfT, wbT):
    return pl.pallas_call(
        _scan_body,
        grid=(_NBLK,),
        in_specs=[
            pl.BlockSpec((_CHUNK, B, G4), lambda i: (i, 0, 0)),
            pl.BlockSpec((_CHUNK, B, G4), lambda i: (_NBLK - 1 - i, 0, 1)),
            pl.BlockSpec((H, G4), lambda i: (0, 0)),
            pl.BlockSpec((H, G4), lambda i: (0, 0)),
        ],
        out_specs=[
            pl.BlockSpec((_CHUNK, B, H), lambda i: (i, 0, 0)),
            pl.BlockSpec((_CHUNK, B, H), lambda i: (_NBLK - 1 - i, 0, 0)),
        ],
        out_shape=[
            jax.ShapeDtypeStruct((S, B, H), jnp.bfloat16),
            jax.ShapeDtypeStruct((S, B, H), jnp.bfloat16),
        ],
        scratch_shapes=[pltpu.VMEM((B, H), jnp.float32) for _ in range(4)],
        compiler_params=pltpu.CompilerParams(
            dimension_semantics=("arbitrary",)),
    )(xp, xp, wfT, wbT)


def _perm_rows(w):
    return jnp.concatenate([w[: 2 * H], w[3 * H:], w[2 * H: 3 * H]], axis=0)


def kernel(input_ids, emb_table, Wih_l0_d0, Whh_l0_d0, bih_l0_d0, bhh_l0_d0,
           Wih_l0_d1, Whh_l0_d1, bih_l0_d1, bhh_l0_d1, Wih_l1_d0, Whh_l1_d0,
           bih_l1_d0, bhh_l1_d0, Wih_l1_d1, Whh_l1_d1, bih_l1_d1, bhh_l1_d1,
           fc_w, fc_b):
    wih = {}
    whh = {}
    bias = {}
    params = {
        (0, 0): (Wih_l0_d0, Whh_l0_d0, bih_l0_d0, bhh_l0_d0),
        (0, 1): (Wih_l0_d1, Whh_l0_d1, bih_l0_d1, bhh_l0_d1),
        (1, 0): (Wih_l1_d0, Whh_l1_d0, bih_l1_d0, bhh_l1_d0),
        (1, 1): (Wih_l1_d1, Whh_l1_d1, bih_l1_d1, bhh_l1_d1),
    }
    for (l, d), (wi, wh, bi, bh) in params.items():
        wih[(l, d)] = _perm_rows(wi).T.astype(jnp.bfloat16)
        whh[(l, d)] = _perm_rows(wh).T.astype(jnp.bfloat16)
        bias[(l, d)] = _perm_rows((bi + bh).reshape(G4, 1)).reshape(G4)

    ids_tm = input_ids.T.reshape(M).astype(jnp.int32)
    x = _emb_gather(emb_table, ids_tm)

    w0 = jnp.concatenate([wih[(0, 0)], wih[(0, 1)]], axis=1)
    b0 = jnp.concatenate([bias[(0, 0)], bias[(0, 1)]])
    xp0 = _mm1(x.astype(jnp.bfloat16), w0, b0,
               jnp.bfloat16).reshape(S, B, 2 * G4)
    hf0, hb0 = _bilstm_scan(xp0, whh[(0, 0)], whh[(0, 1)])

    w1a = jnp.concatenate([wih[(1, 0)][:H], wih[(1, 1)][:H]], axis=1)
    w1b = jnp.concatenate([wih[(1, 0)][H:], wih[(1, 1)][H:]], axis=1)
    b1 = jnp.concatenate([bias[(1, 0)], bias[(1, 1)]])
    xp1 = _mm2(hf0.reshape(M, H), hb0.reshape(M, H), w1a, w1b, b1,
               jnp.bfloat16)
    hf1, hb1 = _bilstm_scan(xp1.reshape(S, B, 2 * G4),
                            whh[(1, 0)], whh[(1, 1)])

    f = fc_w.reshape(T * 5, H2).T
    fpad = jnp.zeros((H2, 128), jnp.float32).at[:, : T * 5].set(f)
    fpad = fpad.astype(jnp.bfloat16)
    bpad = jnp.zeros((128,), jnp.float32).at[: T * 5].set(fc_b.reshape(T * 5))
    y = _mm2(hf1.reshape(M, H), hb1.reshape(M, H),
             fpad[:H], fpad[H:], bpad, jnp.float32)
    logits = y[:, : T * 5].reshape(S, B, T, 5).transpose(1, 2, 0, 3)
    return logits


# speedup vs baseline: 15.2440x; 1.0575x over previous
import functools

import jax
import jax.numpy as jnp
from jax.experimental import pallas as pl
from jax.experimental.pallas import tpu as pltpu
from jax.experimental.pallas import tpu_sc as plsc

V, D, H2, L, T = 30000, 256, 512, 2, 4
H = H2 // 2
B, S = 32, 512
G4 = 4 * H
M = B * S

_WIN = 128


def _emb_gather(table, ids_flat):
    mesh = plsc.VectorSubcoreMesh(core_axis_name="core",
                                  subcore_axis_name="subcore")
    idx2 = ids_flat.reshape(1, M)

    @functools.partial(
        pl.kernel,
        out_type=jax.ShapeDtypeStruct((M, D), jnp.float32),
        mesh=mesh,
    )
    def k(tab_hbm, i_hbm, o_hbm):
        def body(i_vmem, o_vmem):
            pltpu.sync_copy(tab_hbm.at[i_vmem.at[0]], o_vmem)

        pltpu.emit_pipeline(
            body,
            grid=(M // _WIN,),
            in_specs=[pl.BlockSpec((1, _WIN), index_map=lambda i: (0, i))],
            out_specs=[pl.BlockSpec((_WIN, D), index_map=lambda i: (i, 0))],
            core_axis_name="subcore",
            dimension_semantics=(pltpu.PARALLEL,),
        )(i_hbm, o_hbm)

    return k(table, idx2)


_MB = 1024


def _mm1_body(x_ref, w_ref, b_ref, o_ref):
    acc = jnp.dot(x_ref[...], w_ref[...], preferred_element_type=jnp.float32)
    o_ref[...] = (acc + b_ref[...]).astype(o_ref.dtype)


def _mm1(x, w, b, out_dtype):
    m, k = x.shape
    n = w.shape[1]
    return pl.pallas_call(
        _mm1_body,
        grid=(m // _MB,),
        in_specs=[
            pl.BlockSpec((_MB, k), lambda i: (i, 0)),
            pl.BlockSpec((k, n), lambda i: (0, 0)),
            pl.BlockSpec((1, n), lambda i: (0, 0)),
        ],
        out_specs=pl.BlockSpec((_MB, n), lambda i: (i, 0)),
        out_shape=jax.ShapeDtypeStruct((m, n), out_dtype),
        compiler_params=pltpu.CompilerParams(
            dimension_semantics=("parallel",)),
    )(x, w, b.reshape(1, n))


def _mm2_body(xa_ref, xb_ref, wa_ref, wb_ref, b_ref, o_ref):
    acc = jnp.dot(xa_ref[...], wa_ref[...], preferred_element_type=jnp.float32)
    acc = acc + jnp.dot(xb_ref[...], wb_ref[...],
                        preferred_element_type=jnp.float32)
    o_ref[...] = (acc + b_ref[...]).astype(o_ref.dtype)


def _mm2(xa, xb, wa, wb, b, out_dtype):
    m, ka = xa.shape
    kb = xb.shape[1]
    n = wa.shape[1]
    return pl.pallas_call(
        _mm2_body,
        grid=(m // _MB,),
        in_specs=[
            pl.BlockSpec((_MB, ka), lambda i: (i, 0)),
            pl.BlockSpec((_MB, kb), lambda i: (i, 0)),
            pl.BlockSpec((ka, n), lambda i: (0, 0)),
            pl.BlockSpec((kb, n), lambda i: (0, 0)),
            pl.BlockSpec((1, n), lambda i: (0, 0)),
        ],
        out_specs=pl.BlockSpec((_MB, n), lambda i: (i, 0)),
        out_shape=jax.ShapeDtypeStruct((m, n), out_dtype),
        compiler_params=pltpu.CompilerParams(
            dimension_semantics=("parallel",)),
    )(xa, xb, wa, wb, b.reshape(1, n))


_CHUNK = 16
_NBLK = S // _CHUNK


def _lstm_step(x_gates, h, c, w):
    g = x_gates.astype(jnp.float32) + jnp.dot(
        h.astype(jnp.bfloat16), w, preferred_element_type=jnp.float32)
    sg = jax.nn.sigmoid(g[:, : 3 * H])
    gg = jnp.tanh(g[:, 3 * H:])
    c2 = sg[:, H: 2 * H] * c + sg[:, :H] * gg
    h2 = sg[:, 2 * H: 3 * H] * jnp.tanh(c2)
    return h2, c2


def _zero_state(hf_ref, cf_ref, hb_ref, cb_ref):
    @pl.when(pl.program_id(0) == 0)
    def _():
        z = jnp.zeros((B, H), jnp.float32)
        hf_ref[...] = z
        cf_ref[...] = z
        hb_ref[...] = z
        cb_ref[...] = z


def _run_steps(xpf_ref, xpb_ref, wf_ref, wb_ref, of_ref, ob_ref,
               hf_ref, cf_ref, hb_ref, cb_ref):
    wf = wf_ref[...]
    wb = wb_ref[...]
    for j in range(_CHUNK):
        h2, c2 = _lstm_step(xpf_ref[pl.ds(j * B, B)], hf_ref[...],
                            cf_ref[...], wf)
        hf_ref[...] = h2
        cf_ref[...] = c2
        of_ref[j] = h2.astype(jnp.bfloat16)
        jb = _CHUNK - 1 - j
        h2, c2 = _lstm_step(xpb_ref[pl.ds(jb * B, B)], hb_ref[...],
                            cb_ref[...], wb)
        hb_ref[...] = h2
        cb_ref[...] = c2
        ob_ref[jb] = h2.astype(jnp.bfloat16)


def _fscan1_body(xa_ref, xd_ref, win_ref, b_ref, wf_ref, wb_ref,
                 of_ref, ob_ref, xpf_ref, xpb_ref,
                 hf_ref, cf_ref, hb_ref, cb_ref):
    _zero_state(hf_ref, cf_ref, hb_ref, cb_ref)
    xa = xa_ref[...].reshape(_CHUNK * B, D)
    xd = xd_ref[...].reshape(_CHUNK * B, D)
    xpf_ref[...] = jnp.dot(xa, win_ref[:, :G4],
                           preferred_element_type=jnp.float32) + b_ref[:, :G4]
    xpb_ref[...] = jnp.dot(xd, win_ref[:, G4:],
                           preferred_element_type=jnp.float32) + b_ref[:, G4:]
    _run_steps(xpf_ref, xpb_ref, wf_ref, wb_ref, of_ref, ob_ref,
               hf_ref, cf_ref, hb_ref, cb_ref)


def _fused_scan_l0(x, win, b, wfT, wbT):
    return pl.pallas_call(
        _fscan1_body,
        grid=(_NBLK,),
        in_specs=[
            pl.BlockSpec((_CHUNK, B, D), lambda i: (i, 0, 0)),
            pl.BlockSpec((_CHUNK, B, D), lambda i: (_NBLK - 1 - i, 0, 0)),
            pl.BlockSpec((D, 2 * G4), lambda i: (0, 0)),
            pl.BlockSpec((1, 2 * G4), lambda i: (0, 0)),
            pl.BlockSpec((H, G4), lambda i: (0, 0)),
            pl.BlockSpec((H, G4), lambda i: (0, 0)),
        ],
        out_specs=[
            pl.BlockSpec((_CHUNK, B, H), lambda i: (i, 0, 0)),
            pl.BlockSpec((_CHUNK, B, H), lambda i: (_NBLK - 1 - i, 0, 0)),
        ],
        out_shape=[
            jax.ShapeDtypeStruct((S, B, H), jnp.bfloat16),
            jax.ShapeDtypeStruct((S, B, H), jnp.bfloat16),
        ],
        scratch_shapes=[
            pltpu.VMEM((_CHUNK * B, G4), jnp.float32),
            pltpu.VMEM((_CHUNK * B, G4), jnp.float32),
            pltpu.VMEM((B, H), jnp.float32),
            pltpu.VMEM((B, H), jnp.float32),
            pltpu.VMEM((B, H), jnp.float32),
            pltpu.VMEM((B, H), jnp.float32),
        ],
        compiler_params=pltpu.CompilerParams(
            dimension_semantics=("arbitrary",)),
    )(x, x, win, b.reshape(1, 2 * G4), wfT, wbT)


def _fscan2_body(ha_ref, hb0a_ref, hd_ref, hb0d_ref, wina_ref, winb_ref,
                 b_ref, wf_ref, wb_ref, of_ref, ob_ref, xpf_ref, xpb_ref,
                 hf_ref, cf_ref, hb_ref, cb_ref):
    _zero_state(hf_ref, cf_ref, hb_ref, cb_ref)
    ha = ha_ref[...].reshape(_CHUNK * B, H)
    h0a = hb0a_ref[...].reshape(_CHUNK * B, H)
    hd = hd_ref[...].reshape(_CHUNK * B, H)
    h0d = hb0d_ref[...].reshape(_CHUNK * B, H)
    xpf_ref[...] = (
        jnp.dot(ha, wina_ref[:, :G4], preferred_element_type=jnp.float32)
        + jnp.dot(h0a, winb_ref[:, :G4], preferred_element_type=jnp.float32)
        + b_ref[:, :G4])
    xpb_ref[...] = (
        jnp.dot(hd, wina_ref[:, G4:], preferred_element_type=jnp.float32)
        + jnp.dot(h0d, winb_ref[:, G4:], preferred_element_type=jnp.float32)
        + b_ref[:, G4:])
    _run_steps(xpf_ref, xpb_ref, wf_ref, wb_ref, of_ref, ob_ref,
               hf_ref, cf_ref, hb_ref, cb_ref)


def _fused_scan_l1(hf0, hb0, wina, winb, b, wfT, wbT):
    return pl.pallas_call(
        _fscan2_body,
        grid=(_NBLK,),
        in_specs=[
            pl.BlockSpec((_CHUNK, B, H), lambda i: (i, 0, 0)),
            pl.BlockSpec((_CHUNK, B, H), lambda i: (i, 0, 0)),
            pl.BlockSpec((_CHUNK, B, H), lambda i: (_NBLK - 1 - i, 0, 0)),
            pl.BlockSpec((_CHUNK, B, H), lambda i: (_NBLK - 1 - i, 0, 0)),
            pl.BlockSpec((H, 2 * G4), lambda i: (0, 0)),
            pl.BlockSpec((H, 2 * G4), lambda i: (0, 0)),
            pl.BlockSpec((1, 2 * G4), lambda i: (0, 0)),
            pl.BlockSpec((H, G4), lambda i: (0, 0)),
            pl.BlockSpec((H, G4), lambda i: (0, 0)),
        ],
        out_specs=[
            pl.BlockSpec((_CHUNK, B, H), lambda i: (i, 0, 0)),
            pl.BlockSpec((_CHUNK, B, H), lambda i: (_NBLK - 1 - i, 0, 0)),
        ],
        out_shape=[
            jax.ShapeDtypeStruct((S, B, H), jnp.bfloat16),
            jax.ShapeDtypeStruct((S, B, H), jnp.bfloat16),
        ],
        scratch_shapes=[
            pltpu.VMEM((_CHUNK * B, G4), jnp.float32),
            pltpu.VMEM((_CHUNK * B, G4), jnp.float32),
            pltpu.VMEM((B, H), jnp.float32),
            pltpu.VMEM((B, H), jnp.float32),
            pltpu.VMEM((B, H), jnp.float32),
            pltpu.VMEM((B, H), jnp.float32),
        ],
        compiler_params=pltpu.CompilerParams(
            dimension_semantics=("arbitrary",)),
    )(hf0, hb0, hf0, hb0, wina, winb, b.reshape(1, 2 * G4), wfT, wbT)


def _perm_rows(w):
    return jnp.concatenate([w[: 2 * H], w[3 * H:], w[2 * H: 3 * H]], axis=0)


def kernel(input_ids, emb_table, Wih_l0_d0, Whh_l0_d0, bih_l0_d0, bhh_l0_d0,
           Wih_l0_d1, Whh_l0_d1, bih_l0_d1, bhh_l0_d1, Wih_l1_d0, Whh_l1_d0,
           bih_l1_d0, bhh_l1_d0, Wih_l1_d1, Whh_l1_d1, bih_l1_d1, bhh_l1_d1,
           fc_w, fc_b):
    wih = {}
    whh = {}
    bias = {}
    params = {
        (0, 0): (Wih_l0_d0, Whh_l0_d0, bih_l0_d0, bhh_l0_d0),
        (0, 1): (Wih_l0_d1, Whh_l0_d1, bih_l0_d1, bhh_l0_d1),
        (1, 0): (Wih_l1_d0, Whh_l1_d0, bih_l1_d0, bhh_l1_d0),
        (1, 1): (Wih_l1_d1, Whh_l1_d1, bih_l1_d1, bhh_l1_d1),
    }
    for (l, d), (wi, wh, bi, bh) in params.items():
        wih[(l, d)] = _perm_rows(wi).T.astype(jnp.bfloat16)
        whh[(l, d)] = _perm_rows(wh).T.astype(jnp.bfloat16)
        bias[(l, d)] = _perm_rows((bi + bh).reshape(G4, 1)).reshape(G4)

    ids_tm = input_ids.T.reshape(M).astype(jnp.int32)
    x = _emb_gather(emb_table, ids_tm)

    w0 = jnp.concatenate([wih[(0, 0)], wih[(0, 1)]], axis=1)
    b0 = jnp.concatenate([bias[(0, 0)], bias[(0, 1)]])
    x3 = x.astype(jnp.bfloat16).reshape(S, B, D)
    hf0, hb0 = _fused_scan_l0(x3, w0, b0, whh[(0, 0)], whh[(0, 1)])

    w1a = jnp.concatenate([wih[(1, 0)][:H], wih[(1, 1)][:H]], axis=1)
    w1b = jnp.concatenate([wih[(1, 0)][H:], wih[(1, 1)][H:]], axis=1)
    b1 = jnp.concatenate([bias[(1, 0)], bias[(1, 1)]])
    hf1, hb1 = _fused_scan_l1(hf0, hb0, w1a, w1b, b1,
                              whh[(1, 0)], whh[(1, 1)])

    f = fc_w.reshape(T * 5, H2).T
    fpad = jnp.zeros((H2, 128), jnp.float32).at[:, : T * 5].set(f)
    fpad = fpad.astype(jnp.bfloat16)
    bpad = jnp.zeros((128,), jnp.float32).at[: T * 5].set(fc_b.reshape(T * 5))
    y = _mm2(hf1.reshape(M, H), hb1.reshape(M, H),
             fpad[:H], fpad[H:], bpad, jnp.float32)
    logits = y[:, : T * 5].reshape(S, B, T, 5).transpose(1, 2, 0, 3)
    return logits


# speedup vs baseline: 15.6189x; 1.0246x over previous
import functools

import jax
import jax.numpy as jnp
from jax.experimental import pallas as pl
from jax.experimental.pallas import tpu as pltpu
from jax.experimental.pallas import tpu_sc as plsc

V, D, H2, L, T = 30000, 256, 512, 2, 4
H = H2 // 2
B, S = 32, 512
G4 = 4 * H
M = B * S

_WIN = 128


def _emb_gather(table, ids_flat):
    mesh = plsc.VectorSubcoreMesh(core_axis_name="core",
                                  subcore_axis_name="subcore")
    idx2 = ids_flat.reshape(1, M)

    @functools.partial(
        pl.kernel,
        out_type=jax.ShapeDtypeStruct((M, D), jnp.float32),
        mesh=mesh,
    )
    def k(tab_hbm, i_hbm, o_hbm):
        def body(i_vmem, o_vmem):
            pltpu.sync_copy(tab_hbm.at[i_vmem.at[0]], o_vmem)

        pltpu.emit_pipeline(
            body,
            grid=(M // _WIN,),
            in_specs=[pl.BlockSpec((1, _WIN), index_map=lambda i: (0, i))],
            out_specs=[pl.BlockSpec((_WIN, D), index_map=lambda i: (i, 0))],
            core_axis_name=("core", "subcore"),
            dimension_semantics=(pltpu.PARALLEL,),
        )(i_hbm, o_hbm)

    return k(table, idx2)


_CHUNK = 16
_NBLK = S // _CHUNK


def _lstm_step(x_gates, h, c, w):
    g = x_gates.astype(jnp.float32) + jnp.dot(
        h.astype(jnp.bfloat16), w, preferred_element_type=jnp.float32)
    sg = jax.nn.sigmoid(g[:, : 3 * H])
    gg = jnp.tanh(g[:, 3 * H:])
    c2 = sg[:, H: 2 * H] * c + sg[:, :H] * gg
    h2 = sg[:, 2 * H: 3 * H] * jnp.tanh(c2)
    return h2, c2


def _zero_state(hf_ref, cf_ref, hb_ref, cb_ref):
    @pl.when(pl.program_id(0) == 0)
    def _():
        z = jnp.zeros((B, H), jnp.float32)
        hf_ref[...] = z
        cf_ref[...] = z
        hb_ref[...] = z
        cb_ref[...] = z


def _store_h(ref, j, h):
    if len(ref.shape) == 3:
        ref[j] = h.astype(jnp.bfloat16)
    else:
        ref[pl.ds(j * B, B)] = h.astype(jnp.bfloat16)


def _run_steps(xpf_ref, xpb_ref, wf_ref, wb_ref, of_ref, ob_ref,
               hf_ref, cf_ref, hb_ref, cb_ref):
    wf = wf_ref[...]
    wb = wb_ref[...]
    for j in range(_CHUNK):
        h2, c2 = _lstm_step(xpf_ref[pl.ds(j * B, B)], hf_ref[...],
                            cf_ref[...], wf)
        hf_ref[...] = h2
        cf_ref[...] = c2
        _store_h(of_ref, j, h2)
        jb = _CHUNK - 1 - j
        h2, c2 = _lstm_step(xpb_ref[pl.ds(jb * B, B)], hb_ref[...],
                            cb_ref[...], wb)
        hb_ref[...] = h2
        cb_ref[...] = c2
        _store_h(ob_ref, jb, h2)


def _fscan1_body(xa_ref, xd_ref, win_ref, b_ref, wf_ref, wb_ref,
                 of_ref, ob_ref, xpf_ref, xpb_ref,
                 hf_ref, cf_ref, hb_ref, cb_ref):
    _zero_state(hf_ref, cf_ref, hb_ref, cb_ref)
    xa = xa_ref[...].reshape(_CHUNK * B, D)
    xd = xd_ref[...].reshape(_CHUNK * B, D)
    xpf_ref[...] = jnp.dot(xa, win_ref[:, :G4],
                           preferred_element_type=jnp.float32) + b_ref[:, :G4]
    xpb_ref[...] = jnp.dot(xd, win_ref[:, G4:],
                           preferred_element_type=jnp.float32) + b_ref[:, G4:]
    _run_steps(xpf_ref, xpb_ref, wf_ref, wb_ref, of_ref, ob_ref,
               hf_ref, cf_ref, hb_ref, cb_ref)


def _fused_scan_l0(x, win, b, wfT, wbT):
    return pl.pallas_call(
        _fscan1_body,
        grid=(_NBLK,),
        in_specs=[
            pl.BlockSpec((_CHUNK, B, D), lambda i: (i, 0, 0)),
            pl.BlockSpec((_CHUNK, B, D), lambda i: (_NBLK - 1 - i, 0, 0)),
            pl.BlockSpec((D, 2 * G4), lambda i: (0, 0)),
            pl.BlockSpec((1, 2 * G4), lambda i: (0, 0)),
            pl.BlockSpec((H, G4), lambda i: (0, 0)),
            pl.BlockSpec((H, G4), lambda i: (0, 0)),
        ],
        out_specs=[
            pl.BlockSpec((_CHUNK, B, H), lambda i: (i, 0, 0)),
            pl.BlockSpec((_CHUNK, B, H), lambda i: (_NBLK - 1 - i, 0, 0)),
        ],
        out_shape=[
            jax.ShapeDtypeStruct((S, B, H), jnp.bfloat16),
            jax.ShapeDtypeStruct((S, B, H), jnp.bfloat16),
        ],
        scratch_shapes=[
            pltpu.VMEM((_CHUNK * B, G4), jnp.float32),
            pltpu.VMEM((_CHUNK * B, G4), jnp.float32),
            pltpu.VMEM((B, H), jnp.float32),
            pltpu.VMEM((B, H), jnp.float32),
            pltpu.VMEM((B, H), jnp.float32),
            pltpu.VMEM((B, H), jnp.float32),
        ],
        compiler_params=pltpu.CompilerParams(
            dimension_semantics=("arbitrary",)),
    )(x, x, win, b.reshape(1, 2 * G4), wfT, wbT)


def _fscan2_body(ha_ref, hb0a_ref, hd_ref, hb0d_ref, wina_ref, winb_ref,
                 b_ref, wf_ref, wb_ref, fa_ref, fb_ref,
                 yf_ref, yb_ref, xpf_ref, xpb_ref, osf_ref, osb_ref,
                 hf_ref, cf_ref, hb_ref, cb_ref):
    _zero_state(hf_ref, cf_ref, hb_ref, cb_ref)
    ha = ha_ref[...].reshape(_CHUNK * B, H)
    h0a = hb0a_ref[...].reshape(_CHUNK * B, H)
    hd = hd_ref[...].reshape(_CHUNK * B, H)
    h0d = hb0d_ref[...].reshape(_CHUNK * B, H)
    xpf_ref[...] = (
        jnp.dot(ha, wina_ref[:, :G4], preferred_element_type=jnp.float32)
        + jnp.dot(h0a, winb_ref[:, :G4], preferred_element_type=jnp.float32)
        + b_ref[:, :G4])
    xpb_ref[...] = (
        jnp.dot(hd, wina_ref[:, G4:], preferred_element_type=jnp.float32)
        + jnp.dot(h0d, winb_ref[:, G4:], preferred_element_type=jnp.float32)
        + b_ref[:, G4:])
    _run_steps(xpf_ref, xpb_ref, wf_ref, wb_ref, osf_ref, osb_ref,
               hf_ref, cf_ref, hb_ref, cb_ref)
    yf_ref[...] = jnp.dot(
        osf_ref[...], fa_ref[...],
        preferred_element_type=jnp.float32).reshape(_CHUNK, B, 128)
    yb_ref[...] = jnp.dot(
        osb_ref[...], fb_ref[...],
        preferred_element_type=jnp.float32).reshape(_CHUNK, B, 128)


def _fused_scan_l1(hf0, hb0, wina, winb, b, wfT, wbT, fa, fb):
    return pl.pallas_call(
        _fscan2_body,
        grid=(_NBLK,),
        in_specs=[
            pl.BlockSpec((_CHUNK, B, H), lambda i: (i, 0, 0)),
            pl.BlockSpec((_CHUNK, B, H), lambda i: (i, 0, 0)),
            pl.BlockSpec((_CHUNK, B, H), lambda i: (_NBLK - 1 - i, 0, 0)),
            pl.BlockSpec((_CHUNK, B, H), lambda i: (_NBLK - 1 - i, 0, 0)),
            pl.BlockSpec((H, 2 * G4), lambda i: (0, 0)),
            pl.BlockSpec((H, 2 * G4), lambda i: (0, 0)),
            pl.BlockSpec((1, 2 * G4), lambda i: (0, 0)),
            pl.BlockSpec((H, G4), lambda i: (0, 0)),
            pl.BlockSpec((H, G4), lambda i: (0, 0)),
            pl.BlockSpec((H, 128), lambda i: (0, 0)),
            pl.BlockSpec((H, 128), lambda i: (0, 0)),
        ],
        out_specs=[
            pl.BlockSpec((_CHUNK, B, 128), lambda i: (i, 0, 0)),
            pl.BlockSpec((_CHUNK, B, 128), lambda i: (_NBLK - 1 - i, 0, 0)),
        ],
        out_shape=[
            jax.ShapeDtypeStruct((S, B, 128), jnp.float32),
            jax.ShapeDtypeStruct((S, B, 128), jnp.float32),
        ],
        scratch_shapes=[
            pltpu.VMEM((_CHUNK * B, G4), jnp.float32),
            pltpu.VMEM((_CHUNK * B, G4), jnp.float32),
            pltpu.VMEM((_CHUNK * B, H), jnp.bfloat16),
            pltpu.VMEM((_CHUNK * B, H), jnp.bfloat16),
            pltpu.VMEM((B, H), jnp.float32),
            pltpu.VMEM((B, H), jnp.float32),
            pltpu.VMEM((B, H), jnp.float32),
            pltpu.VMEM((B, H), jnp.float32),
        ],
        compiler_params=pltpu.CompilerParams(
            dimension_semantics=("arbitrary",)),
    )(hf0, hb0, hf0, hb0, wina, winb, b.reshape(1, 2 * G4), wfT, wbT, fa, fb)


def _perm_rows(w):
    return jnp.concatenate([w[: 2 * H], w[3 * H:], w[2 * H: 3 * H]], axis=0)


def kernel(input_ids, emb_table, Wih_l0_d0, Whh_l0_d0, bih_l0_d0, bhh_l0_d0,
           Wih_l0_d1, Whh_l0_d1, bih_l0_d1, bhh_l0_d1, Wih_l1_d0, Whh_l1_d0,
           bih_l1_d0, bhh_l1_d0, Wih_l1_d1, Whh_l1_d1, bih_l1_d1, bhh_l1_d1,
           fc_w, fc_b):
    wih = {}
    whh = {}
    bias = {}
    params = {
        (0, 0): (Wih_l0_d0, Whh_l0_d0, bih_l0_d0, bhh_l0_d0),
        (0, 1): (Wih_l0_d1, Whh_l0_d1, bih_l0_d1, bhh_l0_d1),
        (1, 0): (Wih_l1_d0, Whh_l1_d0, bih_l1_d0, bhh_l1_d0),
        (1, 1): (Wih_l1_d1, Whh_l1_d1, bih_l1_d1, bhh_l1_d1),
    }
    for (l, d), (wi, wh, bi, bh) in params.items():
        wih[(l, d)] = _perm_rows(wi).T.astype(jnp.bfloat16)
        whh[(l, d)] = _perm_rows(wh).T.astype(jnp.bfloat16)
        bias[(l, d)] = _perm_rows((bi + bh).reshape(G4, 1)).reshape(G4)

    ids_tm = input_ids.T.reshape(M).astype(jnp.int32)
    x = _emb_gather(emb_table, ids_tm)

    w0 = jnp.concatenate([wih[(0, 0)], wih[(0, 1)]], axis=1)
    b0 = jnp.concatenate([bias[(0, 0)], bias[(0, 1)]])
    x3 = x.astype(jnp.bfloat16).reshape(S, B, D)
    hf0, hb0 = _fused_scan_l0(x3, w0, b0, whh[(0, 0)], whh[(0, 1)])

    w1a = jnp.concatenate([wih[(1, 0)][:H], wih[(1, 1)][:H]], axis=1)
    w1b = jnp.concatenate([wih[(1, 0)][H:], wih[(1, 1)][H:]], axis=1)
    b1 = jnp.concatenate([bias[(1, 0)], bias[(1, 1)]])
    f = fc_w.reshape(T * 5, H2).T
    fpad = jnp.zeros((H2, 128), jnp.float32).at[:, : T * 5].set(f)
    fpad = fpad.astype(jnp.bfloat16)
    yf, yb = _fused_scan_l1(hf0, hb0, w1a, w1b, b1,
                            whh[(1, 0)], whh[(1, 1)], fpad[:H], fpad[H:])
    y = yf[:, :, : T * 5] + yb[:, :, : T * 5] + fc_b.reshape(T * 5)
    logits = y.reshape(S, B, T, 5).transpose(1, 2, 0, 3)
    return logits


# speedup vs baseline: 16.1322x; 1.0329x over previous
import functools

import jax
import jax.numpy as jnp
from jax.experimental import pallas as pl
from jax.experimental.pallas import tpu as pltpu
from jax.experimental.pallas import tpu_sc as plsc

V, D, H2, L, T = 30000, 256, 512, 2, 4
H = H2 // 2
B, S = 32, 512
G4 = 4 * H
M = B * S

_WIN = 128


def _emb_gather(table, ids_flat):
    mesh = plsc.VectorSubcoreMesh(core_axis_name="core",
                                  subcore_axis_name="subcore")
    idx2 = ids_flat.reshape(1, M)

    @functools.partial(
        pl.kernel,
        out_type=jax.ShapeDtypeStruct((M, D), jnp.float32),
        mesh=mesh,
    )
    def k(tab_hbm, i_hbm, o_hbm):
        def body(i_vmem, o_vmem):
            pltpu.sync_copy(tab_hbm.at[i_vmem.at[0]], o_vmem)

        pltpu.emit_pipeline(
            body,
            grid=(M // _WIN,),
            in_specs=[pl.BlockSpec((1, _WIN), index_map=lambda i: (0, i))],
            out_specs=[pl.BlockSpec((_WIN, D), index_map=lambda i: (i, 0))],
            core_axis_name=("core", "subcore"),
            dimension_semantics=(pltpu.PARALLEL,),
        )(i_hbm, o_hbm)

    return k(table, idx2)


_CHUNK = 16
_NBLK = S // _CHUNK


def _lstm_step(x_gates, h, c, w):
    g = x_gates.astype(jnp.float32) + jnp.dot(
        h.astype(jnp.bfloat16), w, preferred_element_type=jnp.float32)
    sif = jax.nn.sigmoid(g[:, : 2 * H])
    gg = jnp.tanh(g[:, 2 * H: 3 * H])
    so = jax.nn.sigmoid(g[:, 3 * H:])
    c2 = sif[:, H:] * c + sif[:, :H] * gg
    h2 = so * jnp.tanh(c2)
    return h2, c2


def _zero_state(hf_ref, cf_ref, hb_ref, cb_ref):
    @pl.when(pl.program_id(0) == 0)
    def _():
        z = jnp.zeros((B, H), jnp.float32)
        hf_ref[...] = z
        cf_ref[...] = z
        hb_ref[...] = z
        cb_ref[...] = z


def _store_h(ref, j, h):
    if len(ref.shape) == 3:
        ref[j] = h.astype(jnp.bfloat16)
    else:
        ref[pl.ds(j * B, B)] = h.astype(jnp.bfloat16)


def _run_steps(xpf_ref, xpb_ref, wf_ref, wb_ref, of_ref, ob_ref,
               hf_ref, cf_ref, hb_ref, cb_ref):
    wf = wf_ref[...]
    wb = wb_ref[...]
    for j in range(_CHUNK):
        h2, c2 = _lstm_step(xpf_ref[pl.ds(j * B, B)], hf_ref[...],
                            cf_ref[...], wf)
        hf_ref[...] = h2
        cf_ref[...] = c2
        _store_h(of_ref, j, h2)
        jb = _CHUNK - 1 - j
        h2, c2 = _lstm_step(xpb_ref[pl.ds(jb * B, B)], hb_ref[...],
                            cb_ref[...], wb)
        hb_ref[...] = h2
        cb_ref[...] = c2
        _store_h(ob_ref, jb, h2)


def _prep_body(wi00_ref, wi01_ref, wi10_ref, wi11_ref,
               wh00_ref, wh01_ref, wh10_ref, wh11_ref,
               bi00_ref, bh00_ref, bi01_ref, bh01_ref,
               bi10_ref, bh10_ref, bi11_ref, bh11_ref, f2_ref,
               w0_ref, w1a_ref, w1b_ref, wf0_ref, wb0_ref, wf1_ref, wb1_ref,
               b0_ref, b1_ref, fa_ref, fb_ref):
    bf = jnp.bfloat16
    w0_ref[:, :G4] = wi00_ref[...].T.astype(bf)
    w0_ref[:, G4:] = wi01_ref[...].T.astype(bf)
    w1a_ref[:, :G4] = wi10_ref[:, :H].T.astype(bf)
    w1a_ref[:, G4:] = wi11_ref[:, :H].T.astype(bf)
    w1b_ref[:, :G4] = wi10_ref[:, H:].T.astype(bf)
    w1b_ref[:, G4:] = wi11_ref[:, H:].T.astype(bf)
    wf0_ref[...] = wh00_ref[...].T.astype(bf)
    wb0_ref[...] = wh01_ref[...].T.astype(bf)
    wf1_ref[...] = wh10_ref[...].T.astype(bf)
    wb1_ref[...] = wh11_ref[...].T.astype(bf)
    b0_ref[:, :G4] = bi00_ref[...] + bh00_ref[...]
    b0_ref[:, G4:] = bi01_ref[...] + bh01_ref[...]
    b1_ref[:, :G4] = bi10_ref[...] + bh10_ref[...]
    b1_ref[:, G4:] = bi11_ref[...] + bh11_ref[...]
    fa_ref[...] = jnp.zeros((H, 32), bf)
    fb_ref[...] = jnp.zeros((H, 32), bf)
    fa_ref[:, : T * 5] = f2_ref[:, :H].T.astype(bf)
    fb_ref[:, : T * 5] = f2_ref[:, H:].T.astype(bf)


def _prep_weights(wi00, wi01, wi10, wi11, wh00, wh01, wh10, wh11,
                  biases, f2):
    ins = [wi00, wi01, wi10, wi11, wh00, wh01, wh10, wh11] + biases + [f2]
    outs = [
        jax.ShapeDtypeStruct((D, 2 * G4), jnp.bfloat16),
        jax.ShapeDtypeStruct((H, 2 * G4), jnp.bfloat16),
        jax.ShapeDtypeStruct((H, 2 * G4), jnp.bfloat16),
        jax.ShapeDtypeStruct((H, G4), jnp.bfloat16),
        jax.ShapeDtypeStruct((H, G4), jnp.bfloat16),
        jax.ShapeDtypeStruct((H, G4), jnp.bfloat16),
        jax.ShapeDtypeStruct((H, G4), jnp.bfloat16),
        jax.ShapeDtypeStruct((1, 2 * G4), jnp.float32),
        jax.ShapeDtypeStruct((1, 2 * G4), jnp.float32),
        jax.ShapeDtypeStruct((H, 32), jnp.bfloat16),
        jax.ShapeDtypeStruct((H, 32), jnp.bfloat16),
    ]
    return pl.pallas_call(_prep_body, out_shape=outs)(*ins)


def _fscan1_body(xa_ref, xd_ref, win_ref, b_ref, wf_ref, wb_ref,
                 of_ref, ob_ref, xpf_ref, xpb_ref,
                 hf_ref, cf_ref, hb_ref, cb_ref):
    _zero_state(hf_ref, cf_ref, hb_ref, cb_ref)
    xa = xa_ref[...].reshape(_CHUNK * B, D).astype(jnp.bfloat16)
    xd = xd_ref[...].reshape(_CHUNK * B, D).astype(jnp.bfloat16)
    xpf_ref[...] = jnp.dot(xa, win_ref[:, :G4],
                           preferred_element_type=jnp.float32) + b_ref[:, :G4]
    xpb_ref[...] = jnp.dot(xd, win_ref[:, G4:],
                           preferred_element_type=jnp.float32) + b_ref[:, G4:]
    _run_steps(xpf_ref, xpb_ref, wf_ref, wb_ref, of_ref, ob_ref,
               hf_ref, cf_ref, hb_ref, cb_ref)


def _fused_scan_l0(x, win, b, wfT, wbT):
    return pl.pallas_call(
        _fscan1_body,
        grid=(_NBLK,),
        in_specs=[
            pl.BlockSpec((_CHUNK, B, D), lambda i: (i, 0, 0)),
            pl.BlockSpec((_CHUNK, B, D), lambda i: (_NBLK - 1 - i, 0, 0)),
            pl.BlockSpec((D, 2 * G4), lambda i: (0, 0)),
            pl.BlockSpec((1, 2 * G4), lambda i: (0, 0)),
            pl.BlockSpec((H, G4), lambda i: (0, 0)),
            pl.BlockSpec((H, G4), lambda i: (0, 0)),
        ],
        out_specs=[
            pl.BlockSpec((_CHUNK, B, H), lambda i: (i, 0, 0)),
            pl.BlockSpec((_CHUNK, B, H), lambda i: (_NBLK - 1 - i, 0, 0)),
        ],
        out_shape=[
            jax.ShapeDtypeStruct((S, B, H), jnp.bfloat16),
            jax.ShapeDtypeStruct((S, B, H), jnp.bfloat16),
        ],
        scratch_shapes=[
            pltpu.VMEM((_CHUNK * B, G4), jnp.float32),
            pltpu.VMEM((_CHUNK * B, G4), jnp.float32),
            pltpu.VMEM((B, H), jnp.float32),
            pltpu.VMEM((B, H), jnp.float32),
            pltpu.VMEM((B, H), jnp.float32),
            pltpu.VMEM((B, H), jnp.float32),
        ],
        compiler_params=pltpu.CompilerParams(
            dimension_semantics=("arbitrary",)),
    )(x, x, win, b.reshape(1, 2 * G4), wfT, wbT)


def _fscan2_body(ha_ref, hb0a_ref, hd_ref, hb0d_ref, wina_ref, winb_ref,
                 b_ref, wf_ref, wb_ref, fa_ref, fb_ref,
                 yf_ref, yb_ref, xpf_ref, xpb_ref, osf_ref, osb_ref,
                 hf_ref, cf_ref, hb_ref, cb_ref):
    _zero_state(hf_ref, cf_ref, hb_ref, cb_ref)
    ha = ha_ref[...].reshape(_CHUNK * B, H)
    h0a = hb0a_ref[...].reshape(_CHUNK * B, H)
    hd = hd_ref[...].reshape(_CHUNK * B, H)
    h0d = hb0d_ref[...].reshape(_CHUNK * B, H)
    xpf_ref[...] = (
        jnp.dot(ha, wina_ref[:, :G4], preferred_element_type=jnp.float32)
        + jnp.dot(h0a, winb_ref[:, :G4], preferred_element_type=jnp.float32)
        + b_ref[:, :G4])
    xpb_ref[...] = (
        jnp.dot(hd, wina_ref[:, G4:], preferred_element_type=jnp.float32)
        + jnp.dot(h0d, winb_ref[:, G4:], preferred_element_type=jnp.float32)
        + b_ref[:, G4:])
    _run_steps(xpf_ref, xpb_ref, wf_ref, wb_ref, osf_ref, osb_ref,
               hf_ref, cf_ref, hb_ref, cb_ref)
    yf_ref[...] = jnp.dot(
        osf_ref[...], fa_ref[...],
        preferred_element_type=jnp.float32).reshape(_CHUNK, B, 32)
    yb_ref[...] = jnp.dot(
        osb_ref[...], fb_ref[...],
        preferred_element_type=jnp.float32).reshape(_CHUNK, B, 32)


def _fused_scan_l1(hf0, hb0, wina, winb, b, wfT, wbT, fa, fb):
    return pl.pallas_call(
        _fscan2_body,
        grid=(_NBLK,),
        in_specs=[
            pl.BlockSpec((_CHUNK, B, H), lambda i: (i, 0, 0)),
            pl.BlockSpec((_CHUNK, B, H), lambda i: (i, 0, 0)),
            pl.BlockSpec((_CHUNK, B, H), lambda i: (_NBLK - 1 - i, 0, 0)),
            pl.BlockSpec((_CHUNK, B, H), lambda i: (_NBLK - 1 - i, 0, 0)),
            pl.BlockSpec((H, 2 * G4), lambda i: (0, 0)),
            pl.BlockSpec((H, 2 * G4), lambda i: (0, 0)),
            pl.BlockSpec((1, 2 * G4), lambda i: (0, 0)),
            pl.BlockSpec((H, G4), lambda i: (0, 0)),
            pl.BlockSpec((H, G4), lambda i: (0, 0)),
            pl.BlockSpec((H, 32), lambda i: (0, 0)),
            pl.BlockSpec((H, 32), lambda i: (0, 0)),
        ],
        out_specs=[
            pl.BlockSpec((_CHUNK, B, 32), lambda i: (i, 0, 0)),
            pl.BlockSpec((_CHUNK, B, 32), lambda i: (_NBLK - 1 - i, 0, 0)),
        ],
        out_shape=[
            jax.ShapeDtypeStruct((S, B, 32), jnp.float32),
            jax.ShapeDtypeStruct((S, B, 32), jnp.float32),
        ],
        scratch_shapes=[
            pltpu.VMEM((_CHUNK * B, G4), jnp.float32),
            pltpu.VMEM((_CHUNK * B, G4), jnp.float32),
            pltpu.VMEM((_CHUNK * B, H), jnp.bfloat16),
            pltpu.VMEM((_CHUNK * B, H), jnp.bfloat16),
            pltpu.VMEM((B, H), jnp.float32),
            pltpu.VMEM((B, H), jnp.float32),
            pltpu.VMEM((B, H), jnp.float32),
            pltpu.VMEM((B, H), jnp.float32),
        ],
        compiler_params=pltpu.CompilerParams(
            dimension_semantics=("arbitrary",)),
    )(hf0, hb0, hf0, hb0, wina, winb, b.reshape(1, 2 * G4), wfT, wbT, fa, fb)


def kernel(input_ids, emb_table, Wih_l0_d0, Whh_l0_d0, bih_l0_d0, bhh_l0_d0,
           Wih_l0_d1, Whh_l0_d1, bih_l0_d1, bhh_l0_d1, Wih_l1_d0, Whh_l1_d0,
           bih_l1_d0, bhh_l1_d0, Wih_l1_d1, Whh_l1_d1, bih_l1_d1, bhh_l1_d1,
           fc_w, fc_b):
    biases = [b.reshape(1, G4) for b in
              (bih_l0_d0, bhh_l0_d0, bih_l0_d1, bhh_l0_d1,
               bih_l1_d0, bhh_l1_d0, bih_l1_d1, bhh_l1_d1)]
    (w0, w1a, w1b, wf0, wb0, wf1, wb1, b0, b1, fa, fb) = _prep_weights(
        Wih_l0_d0, Wih_l0_d1, Wih_l1_d0, Wih_l1_d1,
        Whh_l0_d0, Whh_l0_d1, Whh_l1_d0, Whh_l1_d1,
        biases, fc_w.reshape(T * 5, H2))

    ids_tm = input_ids.T.reshape(M).astype(jnp.int32)
    x = _emb_gather(emb_table, ids_tm)

    hf0, hb0 = _fused_scan_l0(x.reshape(S, B, D), w0, b0, wf0, wb0)

    yf, yb = _fused_scan_l1(hf0, hb0, w1a, w1b, b1, wf1, wb1, fa, fb)
    y = yf[:, :, : T * 5] + yb[:, :, : T * 5] + fc_b.reshape(T * 5)
    logits = y.reshape(S, B, T, 5).transpose(1, 2, 0, 3)
    return logits


# speedup vs baseline: 16.3535x; 1.0137x over previous
import functools

import jax
import jax.numpy as jnp
from jax.experimental import pallas as pl
from jax.experimental.pallas import tpu as pltpu
from jax.experimental.pallas import tpu_sc as plsc

V, D, H2, L, T = 30000, 256, 512, 2, 4
H = H2 // 2
B, S = 32, 512
G4 = 4 * H
M = B * S

_WIN = 128


def _emb_gather(table, ids_flat):
    mesh = plsc.VectorSubcoreMesh(core_axis_name="core",
                                  subcore_axis_name="subcore")
    idx2 = ids_flat.reshape(1, M)

    @functools.partial(
        pl.kernel,
        out_type=jax.ShapeDtypeStruct((M, D), jnp.float32),
        mesh=mesh,
    )
    def k(tab_hbm, i_hbm, o_hbm):
        def body(i_vmem, o_vmem):
            pltpu.sync_copy(tab_hbm.at[i_vmem.at[0]], o_vmem)

        pltpu.emit_pipeline(
            body,
            grid=(M // _WIN,),
            in_specs=[pl.BlockSpec((1, _WIN), index_map=lambda i: (0, i))],
            out_specs=[pl.BlockSpec((_WIN, D), index_map=lambda i: (i, 0))],
            core_axis_name=("core", "subcore"),
            dimension_semantics=(pltpu.PARALLEL,),
        )(i_hbm, o_hbm)

    return k(table, idx2)


_CHUNK = 32
_NBLK = S // _CHUNK


def _lstm_step(x_gates, h, c, w):
    g = x_gates.astype(jnp.float32) + jnp.dot(
        h.astype(jnp.bfloat16), w, preferred_element_type=jnp.float32)
    sif = jax.nn.sigmoid(g[:, : 2 * H])
    gg = jnp.tanh(g[:, 2 * H: 3 * H])
    so = jax.nn.sigmoid(g[:, 3 * H:])
    c2 = sif[:, H:] * c + sif[:, :H] * gg
    h2 = so * jnp.tanh(c2)
    return h2, c2


def _zero_state(hf_ref, cf_ref, hb_ref, cb_ref):
    @pl.when(pl.program_id(0) == 0)
    def _():
        z = jnp.zeros((B, H), jnp.float32)
        hf_ref[...] = z
        cf_ref[...] = z
        hb_ref[...] = z
        cb_ref[...] = z


def _store_h(ref, j, h):
    if len(ref.shape) == 3:
        ref[j] = h.astype(jnp.bfloat16)
    else:
        ref[pl.ds(j * B, B)] = h.astype(jnp.bfloat16)


def _run_steps(xpf_ref, xpb_ref, wf_ref, wb_ref, of_ref, ob_ref,
               hf_ref, cf_ref, hb_ref, cb_ref):
    wf = wf_ref[...]
    wb = wb_ref[...]
    for j in range(_CHUNK):
        h2, c2 = _lstm_step(xpf_ref[pl.ds(j * B, B)], hf_ref[...],
                            cf_ref[...], wf)
        hf_ref[...] = h2
        cf_ref[...] = c2
        _store_h(of_ref, j, h2)
        jb = _CHUNK - 1 - j
        h2, c2 = _lstm_step(xpb_ref[pl.ds(jb * B, B)], hb_ref[...],
                            cb_ref[...], wb)
        hb_ref[...] = h2
        cb_ref[...] = c2
        _store_h(ob_ref, jb, h2)


def _prep_body(wi00_ref, wi01_ref, wi10_ref, wi11_ref,
               wh00_ref, wh01_ref, wh10_ref, wh11_ref,
               bi00_ref, bh00_ref, bi01_ref, bh01_ref,
               bi10_ref, bh10_ref, bi11_ref, bh11_ref, f2_ref,
               w0_ref, w1a_ref, w1b_ref, wf0_ref, wb0_ref, wf1_ref, wb1_ref,
               b0_ref, b1_ref, fa_ref, fb_ref):
    bf = jnp.bfloat16
    w0_ref[:, :G4] = wi00_ref[...].T.astype(bf)
    w0_ref[:, G4:] = wi01_ref[...].T.astype(bf)
    w1a_ref[:, :G4] = wi10_ref[:, :H].T.astype(bf)
    w1a_ref[:, G4:] = wi11_ref[:, :H].T.astype(bf)
    w1b_ref[:, :G4] = wi10_ref[:, H:].T.astype(bf)
    w1b_ref[:, G4:] = wi11_ref[:, H:].T.astype(bf)
    wf0_ref[...] = wh00_ref[...].T.astype(bf)
    wb0_ref[...] = wh01_ref[...].T.astype(bf)
    wf1_ref[...] = wh10_ref[...].T.astype(bf)
    wb1_ref[...] = wh11_ref[...].T.astype(bf)
    b0_ref[:, :G4] = bi00_ref[...] + bh00_ref[...]
    b0_ref[:, G4:] = bi01_ref[...] + bh01_ref[...]
    b1_ref[:, :G4] = bi10_ref[...] + bh10_ref[...]
    b1_ref[:, G4:] = bi11_ref[...] + bh11_ref[...]
    fa_ref[...] = jnp.zeros((H, 32), bf)
    fb_ref[...] = jnp.zeros((H, 32), bf)
    fa_ref[:, : T * 5] = f2_ref[:, :H].T.astype(bf)
    fb_ref[:, : T * 5] = f2_ref[:, H:].T.astype(bf)


def _prep_weights(wi00, wi01, wi10, wi11, wh00, wh01, wh10, wh11,
                  biases, f2):
    ins = [wi00, wi01, wi10, wi11, wh00, wh01, wh10, wh11] + biases + [f2]
    outs = [
        jax.ShapeDtypeStruct((D, 2 * G4), jnp.bfloat16),
        jax.ShapeDtypeStruct((H, 2 * G4), jnp.bfloat16),
        jax.ShapeDtypeStruct((H, 2 * G4), jnp.bfloat16),
        jax.ShapeDtypeStruct((H, G4), jnp.bfloat16),
        jax.ShapeDtypeStruct((H, G4), jnp.bfloat16),
        jax.ShapeDtypeStruct((H, G4), jnp.bfloat16),
        jax.ShapeDtypeStruct((H, G4), jnp.bfloat16),
        jax.ShapeDtypeStruct((1, 2 * G4), jnp.float32),
        jax.ShapeDtypeStruct((1, 2 * G4), jnp.float32),
        jax.ShapeDtypeStruct((H, 32), jnp.bfloat16),
        jax.ShapeDtypeStruct((H, 32), jnp.bfloat16),
    ]
    return pl.pallas_call(_prep_body, out_shape=outs)(*ins)


def _fscan1_body(xa_ref, xd_ref, win_ref, b_ref, wf_ref, wb_ref,
                 of_ref, ob_ref, xpf_ref, xpb_ref,
                 hf_ref, cf_ref, hb_ref, cb_ref):
    _zero_state(hf_ref, cf_ref, hb_ref, cb_ref)
    xa = xa_ref[...].reshape(_CHUNK * B, D).astype(jnp.bfloat16)
    xd = xd_ref[...].reshape(_CHUNK * B, D).astype(jnp.bfloat16)
    xpf_ref[...] = jnp.dot(xa, win_ref[:, :G4],
                           preferred_element_type=jnp.float32) + b_ref[:, :G4]
    xpb_ref[...] = jnp.dot(xd, win_ref[:, G4:],
                           preferred_element_type=jnp.float32) + b_ref[:, G4:]
    _run_steps(xpf_ref, xpb_ref, wf_ref, wb_ref, of_ref, ob_ref,
               hf_ref, cf_ref, hb_ref, cb_ref)


def _fused_scan_l0(x, win, b, wfT, wbT):
    return pl.pallas_call(
        _fscan1_body,
        grid=(_NBLK,),
        in_specs=[
            pl.BlockSpec((_CHUNK, B, D), lambda i: (i, 0, 0)),
            pl.BlockSpec((_CHUNK, B, D), lambda i: (_NBLK - 1 - i, 0, 0)),
            pl.BlockSpec((D, 2 * G4), lambda i: (0, 0)),
            pl.BlockSpec((1, 2 * G4), lambda i: (0, 0)),
            pl.BlockSpec((H, G4), lambda i: (0, 0)),
            pl.BlockSpec((H, G4), lambda i: (0, 0)),
        ],
        out_specs=[
            pl.BlockSpec((_CHUNK, B, H), lambda i: (i, 0, 0)),
            pl.BlockSpec((_CHUNK, B, H), lambda i: (_NBLK - 1 - i, 0, 0)),
        ],
        out_shape=[
            jax.ShapeDtypeStruct((S, B, H), jnp.bfloat16),
            jax.ShapeDtypeStruct((S, B, H), jnp.bfloat16),
        ],
        scratch_shapes=[
            pltpu.VMEM((_CHUNK * B, G4), jnp.float32),
            pltpu.VMEM((_CHUNK * B, G4), jnp.float32),
            pltpu.VMEM((B, H), jnp.float32),
            pltpu.VMEM((B, H), jnp.float32),
            pltpu.VMEM((B, H), jnp.float32),
            pltpu.VMEM((B, H), jnp.float32),
        ],
        compiler_params=pltpu.CompilerParams(
            dimension_semantics=("arbitrary",)),
    )(x, x, win, b.reshape(1, 2 * G4), wfT, wbT)


def _fscan2_body(ha_ref, hb0a_ref, hd_ref, hb0d_ref, wina_ref, winb_ref,
                 b_ref, wf_ref, wb_ref, fa_ref, fb_ref,
                 yf_ref, yb_ref, xpf_ref, xpb_ref, osf_ref, osb_ref,
                 hf_ref, cf_ref, hb_ref, cb_ref):
    _zero_state(hf_ref, cf_ref, hb_ref, cb_ref)
    ha = ha_ref[...].reshape(_CHUNK * B, H)
    h0a = hb0a_ref[...].reshape(_CHUNK * B, H)
    hd = hd_ref[...].reshape(_CHUNK * B, H)
    h0d = hb0d_ref[...].reshape(_CHUNK * B, H)
    xpf_ref[...] = (
        jnp.dot(ha, wina_ref[:, :G4], preferred_element_type=jnp.float32)
        + jnp.dot(h0a, winb_ref[:, :G4], preferred_element_type=jnp.float32)
        + b_ref[:, :G4])
    xpb_ref[...] = (
        jnp.dot(hd, wina_ref[:, G4:], preferred_element_type=jnp.float32)
        + jnp.dot(h0d, winb_ref[:, G4:], preferred_element_type=jnp.float32)
        + b_ref[:, G4:])
    _run_steps(xpf_ref, xpb_ref, wf_ref, wb_ref, osf_ref, osb_ref,
               hf_ref, cf_ref, hb_ref, cb_ref)
    yf_ref[...] = jnp.dot(
        osf_ref[...], fa_ref[...],
        preferred_element_type=jnp.float32).reshape(_CHUNK, B, 32)
    yb_ref[...] = jnp.dot(
        osb_ref[...], fb_ref[...],
        preferred_element_type=jnp.float32).reshape(_CHUNK, B, 32)


def _fused_scan_l1(hf0, hb0, wina, winb, b, wfT, wbT, fa, fb):
    return pl.pallas_call(
        _fscan2_body,
        grid=(_NBLK,),
        in_specs=[
            pl.BlockSpec((_CHUNK, B, H), lambda i: (i, 0, 0)),
            pl.BlockSpec((_CHUNK, B, H), lambda i: (i, 0, 0)),
            pl.BlockSpec((_CHUNK, B, H), lambda i: (_NBLK - 1 - i, 0, 0)),
            pl.BlockSpec((_CHUNK, B, H), lambda i: (_NBLK - 1 - i, 0, 0)),
            pl.BlockSpec((H, 2 * G4), lambda i: (0, 0)),
            pl.BlockSpec((H, 2 * G4), lambda i: (0, 0)),
            pl.BlockSpec((1, 2 * G4), lambda i: (0, 0)),
            pl.BlockSpec((H, G4), lambda i: (0, 0)),
            pl.BlockSpec((H, G4), lambda i: (0, 0)),
            pl.BlockSpec((H, 32), lambda i: (0, 0)),
            pl.BlockSpec((H, 32), lambda i: (0, 0)),
        ],
        out_specs=[
            pl.BlockSpec((_CHUNK, B, 32), lambda i: (i, 0, 0)),
            pl.BlockSpec((_CHUNK, B, 32), lambda i: (_NBLK - 1 - i, 0, 0)),
        ],
        out_shape=[
            jax.ShapeDtypeStruct((S, B, 32), jnp.float32),
            jax.ShapeDtypeStruct((S, B, 32), jnp.float32),
        ],
        scratch_shapes=[
            pltpu.VMEM((_CHUNK * B, G4), jnp.float32),
            pltpu.VMEM((_CHUNK * B, G4), jnp.float32),
            pltpu.VMEM((_CHUNK * B, H), jnp.bfloat16),
            pltpu.VMEM((_CHUNK * B, H), jnp.bfloat16),
            pltpu.VMEM((B, H), jnp.float32),
            pltpu.VMEM((B, H), jnp.float32),
            pltpu.VMEM((B, H), jnp.float32),
            pltpu.VMEM((B, H), jnp.float32),
        ],
        compiler_params=pltpu.CompilerParams(
            dimension_semantics=("arbitrary",)),
    )(hf0, hb0, hf0, hb0, wina, winb, b.reshape(1, 2 * G4), wfT, wbT, fa, fb)


def kernel(input_ids, emb_table, Wih_l0_d0, Whh_l0_d0, bih_l0_d0, bhh_l0_d0,
           Wih_l0_d1, Whh_l0_d1, bih_l0_d1, bhh_l0_d1, Wih_l1_d0, Whh_l1_d0,
           bih_l1_d0, bhh_l1_d0, Wih_l1_d1, Whh_l1_d1, bih_l1_d1, bhh_l1_d1,
           fc_w, fc_b):
    biases = [b.reshape(1, G4) for b in
              (bih_l0_d0, bhh_l0_d0, bih_l0_d1, bhh_l0_d1,
               bih_l1_d0, bhh_l1_d0, bih_l1_d1, bhh_l1_d1)]
    (w0, w1a, w1b, wf0, wb0, wf1, wb1, b0, b1, fa, fb) = _prep_weights(
        Wih_l0_d0, Wih_l0_d1, Wih_l1_d0, Wih_l1_d1,
        Whh_l0_d0, Whh_l0_d1, Whh_l1_d0, Whh_l1_d1,
        biases, fc_w.reshape(T * 5, H2))

    ids_tm = input_ids.T.reshape(M).astype(jnp.int32)
    x = _emb_gather(emb_table, ids_tm)

    hf0, hb0 = _fused_scan_l0(x.reshape(S, B, D), w0, b0, wf0, wb0)

    yf, yb = _fused_scan_l1(hf0, hb0, w1a, w1b, b1, wf1, wb1, fa, fb)
    y = yf[:, :, : T * 5] + yb[:, :, : T * 5] + fc_b.reshape(T * 5)
    logits = y.reshape(S, B, T, 5).transpose(1, 2, 0, 3)
    return logits
